# Initial kernel scaffold; baseline (speedup 1.0000x reference)
import jax
import jax.numpy as jnp
from jax import lax
from jax.experimental import pallas as pl
from jax.experimental.pallas import tpu as pltpu
from jax.experimental.pallas import tpu_sc as plsc

N = 10000
D = 128
E = 320000
G = 64

NC = 2
NS = 16
NW = NC * NS
CHUNK = 128
CPT = -(-E // (NW * CHUNK))
EPAD = NW * CPT * CHUNK
ROWS_PER_TILE_INIT = -(-(N + 1) // NS)
AGG_ROWS = ROWS_PER_TILE_INIT * NS
ROWS_PER_TILE_OUT = N // NS

BLK = 1000
NBLK = N // BLK


def _sc_agg_body(h_hbm, src_hbm, dst_hbm, zeros_hbm,
                 out0_hbm, out1_hbm,
                 src_v, dst_v, rows_v, agg_sh, sem):
    c = lax.axis_index("c")
    s = lax.axis_index("s")
    wid = s * NC + c

    pltpu.sync_copy(zeros_hbm, agg_sh.at[pl.ds(s * ROWS_PER_TILE_INIT,
                                               ROWS_PER_TILE_INIT)])
    pltpu.sync_copy(src_hbm.at[wid], src_v)
    pltpu.sync_copy(dst_hbm.at[wid], dst_v)
    plsc.subcore_barrier()

    def step(i, carry):
        pltpu.async_copy(h_hbm.at[src_v.at[i]], rows_v, sem).wait()
        pltpu.sync_copy(rows_v, agg_sh.at[dst_v.at[i]], add=True)
        return carry

    lax.fori_loop(0, CPT, step, 0)
    plsc.subcore_barrier()

    row0 = s * ROWS_PER_TILE_OUT

    @pl.when(c == 0)
    def _():
        pltpu.sync_copy(agg_sh.at[pl.ds(row0, ROWS_PER_TILE_OUT)],
                        out0_hbm.at[pl.ds(row0, ROWS_PER_TILE_OUT)])

    @pl.when(c == 1)
    def _():
        pltpu.sync_copy(agg_sh.at[pl.ds(row0, ROWS_PER_TILE_OUT)],
                        out1_hbm.at[pl.ds(row0, ROWS_PER_TILE_OUT)])


_sc_aggregate = pl.kernel(
    _sc_agg_body,
    out_type=(jax.ShapeDtypeStruct((N, D), jnp.float32),
              jax.ShapeDtypeStruct((N, D), jnp.float32)),
    mesh=plsc.VectorSubcoreMesh(core_axis_name="c", subcore_axis_name="s"),
    scratch_types=[
        pltpu.VMEM((CPT, CHUNK), jnp.int32),
        pltpu.VMEM((CPT, CHUNK), jnp.int32),
        pltpu.VMEM((CHUNK, D), jnp.float32),
        pltpu.VMEM_SHARED((AGG_ROWS, D), jnp.float32),
        pltpu.SemaphoreType.DMA,
    ],
)


def _mlp_body(x_ref, p0_ref, p1_ref, w1_ref, b1_ref, w2_ref, b2_ref,
              hpre_ref, stats_ref):
    i = pl.program_id(0)
    u = x_ref[...] + p0_ref[...] + p1_ref[...]
    a = jnp.maximum(
        jnp.dot(u, w1_ref[...], preferred_element_type=jnp.float32)
        + b1_ref[...], 0.0)
    hp = (jnp.dot(a, w2_ref[...], preferred_element_type=jnp.float32)
          + b2_ref[...])
    hpre_ref[...] = hp
    part = jnp.concatenate([jnp.sum(hp, 0, keepdims=True),
                            jnp.sum(hp * hp, 0, keepdims=True)], axis=0)

    @pl.when(i == 0)
    def _():
        stats_ref[...] = part

    @pl.when(i > 0)
    def _():
        stats_ref[...] = stats_ref[...] + part


_mlp_call = pl.pallas_call(
    _mlp_body,
    grid=(NBLK,),
    in_specs=[
        pl.BlockSpec((BLK, D), lambda i: (i, 0)),
        pl.BlockSpec((BLK, D), lambda i: (i, 0)),
        pl.BlockSpec((BLK, D), lambda i: (i, 0)),
        pl.BlockSpec((D, D), lambda i: (0, 0)),
        pl.BlockSpec((1, D), lambda i: (0, 0)),
        pl.BlockSpec((D, D), lambda i: (0, 0)),
        pl.BlockSpec((1, D), lambda i: (0, 0)),
    ],
    out_specs=[
        pl.BlockSpec((BLK, D), lambda i: (i, 0)),
        pl.BlockSpec((2, D), lambda i: (0, 0)),
    ],
    out_shape=[
        jax.ShapeDtypeStruct((N, D), jnp.float32),
        jax.ShapeDtypeStruct((2, D), jnp.float32),
    ],
)


def _bn_stats(stats):
    mean = stats[0:1, :] * (1.0 / N)
    var = stats[1:2, :] * (1.0 / N) - mean * mean
    inv = lax.rsqrt(var + 1e-5)
    return mean, inv


def _bn_body(hp_ref, stats_ref, g_ref, b_ref, out_ref):
    mean, inv = _bn_stats(stats_ref[...])
    out_ref[...] = jnp.maximum(
        (hp_ref[...] - mean) * inv * g_ref[...] + b_ref[...], 0.0)


_bn_call = pl.pallas_call(
    _bn_body,
    grid=(NBLK,),
    in_specs=[
        pl.BlockSpec((BLK, D), lambda i: (i, 0)),
        pl.BlockSpec((2, D), lambda i: (0, 0)),
        pl.BlockSpec((1, D), lambda i: (0, 0)),
        pl.BlockSpec((1, D), lambda i: (0, 0)),
    ],
    out_specs=pl.BlockSpec((BLK, D), lambda i: (i, 0)),
    out_shape=jax.ShapeDtypeStruct((N, D), jnp.float32),
)


def _pool_body(hp_ref, stats_ref, g_ref, b_ref, bidx_ref,
               pw1_ref, pb1_ref, pw2_ref, pb2_ref,
               out_ref, accp_ref, accc_ref):
    i = pl.program_id(0)
    mean, inv = _bn_stats(stats_ref[...])
    h = jnp.maximum(
        (hp_ref[...] - mean) * inv * g_ref[...] + b_ref[...], 0.0)
    onehot_t = (lax.broadcasted_iota(jnp.int32, (G, BLK), 0)
                == bidx_ref[...]).astype(jnp.float32)
    pp = jnp.dot(onehot_t, h, preferred_element_type=jnp.float32)
    cnt = jnp.sum(onehot_t, axis=1, keepdims=True)
    pc = jnp.broadcast_to(cnt, (G, D))

    @pl.when(i == 0)
    def _():
        accp_ref[...] = pp
        accc_ref[...] = pc

    @pl.when(i > 0)
    def _():
        accp_ref[...] = accp_ref[...] + pp
        accc_ref[...] = accc_ref[...] + pc

    @pl.when(i == NBLK - 1)
    def _():
        pooled = accp_ref[...] / jnp.maximum(accc_ref[...], 1.0)
        a = jnp.maximum(
            jnp.dot(pooled, pw1_ref[...], preferred_element_type=jnp.float32)
            + pb1_ref[...], 0.0)
        out_ref[...] = (jnp.dot(a, pw2_ref[...],
                                preferred_element_type=jnp.float32)
                        + pb2_ref[...])


def _make_pool_call(T):
    return pl.pallas_call(
        _pool_body,
        grid=(NBLK,),
        in_specs=[
            pl.BlockSpec((BLK, D), lambda i: (i, 0)),
            pl.BlockSpec((2, D), lambda i: (0, 0)),
            pl.BlockSpec((1, D), lambda i: (0, 0)),
            pl.BlockSpec((1, D), lambda i: (0, 0)),
            pl.BlockSpec((1, BLK), lambda i: (i, 0)),
            pl.BlockSpec((D, D), lambda i: (0, 0)),
            pl.BlockSpec((1, D), lambda i: (0, 0)),
            pl.BlockSpec((D, T), lambda i: (0, 0)),
            pl.BlockSpec((1, T), lambda i: (0, 0)),
        ],
        out_specs=pl.BlockSpec((G, T), lambda i: (0, 0)),
        out_shape=jax.ShapeDtypeStruct((G, T), jnp.float32),
        scratch_shapes=[
            pltpu.VMEM((G, D), jnp.float32),
            pltpu.VMEM((G, D), jnp.float32),
        ],
    )


def kernel(x, edge_index, batch_idx,
           l0_W1, l0_b1, l0_W2, l0_b2, l0_gamma, l0_beta,
           l1_W1, l1_b1, l1_W2, l1_b2, l1_gamma, l1_beta,
           l2_W1, l2_b1, l2_W2, l2_b2, l2_gamma, l2_beta,
           p_W1, p_b1, p_W2, p_b2):
    T = p_W2.shape[1]
    src = edge_index[0]
    dst = edge_index[1]
    pad = EPAD - E
    src_p = jnp.concatenate(
        [src, jnp.zeros((pad,), jnp.int32)]).reshape(NW, CPT, CHUNK)
    dst_p = jnp.concatenate(
        [dst, jnp.full((pad,), N, jnp.int32)]).reshape(NW, CPT, CHUNK)
    zeros_blk = jnp.zeros((ROWS_PER_TILE_INIT, D), jnp.float32)
    bidx = batch_idx.reshape(NBLK, BLK)

    layers = (
        (l0_W1, l0_b1, l0_W2, l0_b2, l0_gamma, l0_beta),
        (l1_W1, l1_b1, l1_W2, l1_b2, l1_gamma, l1_beta),
        (l2_W1, l2_b1, l2_W2, l2_b2, l2_gamma, l2_beta),
    )

    h = x
    for li, (W1, b1, W2, b2, gamma, beta) in enumerate(layers):
        p0, p1 = _sc_aggregate(h, src_p, dst_p, zeros_blk)
        hpre, stats = _mlp_call(h, p0, p1,
                                W1, b1.reshape(1, D), W2, b2.reshape(1, D))
        if li < 2:
            h = _bn_call(hpre, stats,
                         gamma.reshape(1, D), beta.reshape(1, D))
    out = _make_pool_call(T)(
        hpre, stats, l2_gamma.reshape(1, D), l2_beta.reshape(1, D), bidx,
        p_W1, p_b1.reshape(1, D), p_W2, p_b2.reshape(1, T))
    return out


# speedup vs baseline: 4.3694x; 4.3694x over previous
import jax
import jax.numpy as jnp
from jax import lax
from jax.experimental import pallas as pl
from jax.experimental.pallas import tpu as pltpu
from jax.experimental.pallas import tpu_sc as plsc

N = 10000
D = 128
E = 320000
G = 64

NC = 2
NS = 16
NW = NC * NS
CHUNK = 128
CPT = -(-E // (NW * CHUNK))
EPAD = NW * CPT * CHUNK
ROWS_PER_TILE = 632
AGG_ROWS = ROWS_PER_TILE * NS

BLK = 1000
NBLK = N // BLK


def _sc_agg_body(h_hbm, src_hbm, dst_hbm, zeros_hbm,
                 out0_hbm, out1_hbm,
                 src_v, dst_v, rows_v, agg_sh, sem):
    c = lax.axis_index("c")
    s = lax.axis_index("s")
    wid = s * NC + c

    pltpu.sync_copy(zeros_hbm, agg_sh.at[pl.ds(s * ROWS_PER_TILE,
                                               ROWS_PER_TILE)])
    pltpu.sync_copy(src_hbm.at[wid], src_v)
    pltpu.sync_copy(dst_hbm.at[wid], dst_v)
    plsc.subcore_barrier()

    def step(i, carry):
        pltpu.async_copy(h_hbm.at[src_v.at[i]], rows_v, sem).wait()
        pltpu.sync_copy(rows_v, agg_sh.at[dst_v.at[i]], add=True)
        return carry

    lax.fori_loop(0, CPT, step, 0)
    plsc.subcore_barrier()

    row0 = s * ROWS_PER_TILE

    @pl.when(c == 0)
    def _():
        pltpu.sync_copy(agg_sh.at[pl.ds(row0, ROWS_PER_TILE)],
                        out0_hbm.at[pl.ds(row0, ROWS_PER_TILE)])

    @pl.when(c == 1)
    def _():
        pltpu.sync_copy(agg_sh.at[pl.ds(row0, ROWS_PER_TILE)],
                        out1_hbm.at[pl.ds(row0, ROWS_PER_TILE)])


_sc_aggregate = pl.kernel(
    _sc_agg_body,
    out_type=(jax.ShapeDtypeStruct((AGG_ROWS, D), jnp.float32),
              jax.ShapeDtypeStruct((AGG_ROWS, D), jnp.float32)),
    mesh=plsc.VectorSubcoreMesh(core_axis_name="c", subcore_axis_name="s"),
    scratch_types=[
        pltpu.VMEM((CPT, CHUNK), jnp.int32),
        pltpu.VMEM((CPT, CHUNK), jnp.int32),
        pltpu.VMEM((CHUNK, D), jnp.float32),
        pltpu.VMEM_SHARED((AGG_ROWS, D), jnp.float32),
        pltpu.SemaphoreType.DMA,
    ],
)


def _mlp_body(x_ref, p0_ref, p1_ref, w1_ref, b1_ref, w2_ref, b2_ref,
              hpre_ref, stats_ref):
    i = pl.program_id(0)
    u = x_ref[...] + p0_ref[...] + p1_ref[...]
    a = jnp.maximum(
        jnp.dot(u, w1_ref[...], preferred_element_type=jnp.float32)
        + b1_ref[...], 0.0)
    hp = (jnp.dot(a, w2_ref[...], preferred_element_type=jnp.float32)
          + b2_ref[...])
    hpre_ref[...] = hp
    part = jnp.concatenate([jnp.sum(hp, 0, keepdims=True),
                            jnp.sum(hp * hp, 0, keepdims=True)], axis=0)

    @pl.when(i == 0)
    def _():
        stats_ref[...] = part

    @pl.when(i > 0)
    def _():
        stats_ref[...] = stats_ref[...] + part


_mlp_call = pl.pallas_call(
    _mlp_body,
    grid=(NBLK,),
    in_specs=[
        pl.BlockSpec((BLK, D), lambda i: (i, 0)),
        pl.BlockSpec((BLK, D), lambda i: (i, 0)),
        pl.BlockSpec((BLK, D), lambda i: (i, 0)),
        pl.BlockSpec((D, D), lambda i: (0, 0)),
        pl.BlockSpec((1, D), lambda i: (0, 0)),
        pl.BlockSpec((D, D), lambda i: (0, 0)),
        pl.BlockSpec((1, D), lambda i: (0, 0)),
    ],
    out_specs=[
        pl.BlockSpec((BLK, D), lambda i: (i, 0)),
        pl.BlockSpec((2, D), lambda i: (0, 0)),
    ],
    out_shape=[
        jax.ShapeDtypeStruct((N, D), jnp.float32),
        jax.ShapeDtypeStruct((2, D), jnp.float32),
    ],
)


def _bn_stats(stats):
    mean = stats[0:1, :] * (1.0 / N)
    var = stats[1:2, :] * (1.0 / N) - mean * mean
    inv = lax.rsqrt(var + 1e-5)
    return mean, inv


def _bn_body(hp_ref, stats_ref, g_ref, b_ref, out_ref):
    mean, inv = _bn_stats(stats_ref[...])
    out_ref[...] = jnp.maximum(
        (hp_ref[...] - mean) * inv * g_ref[...] + b_ref[...], 0.0)


_bn_call = pl.pallas_call(
    _bn_body,
    grid=(NBLK,),
    in_specs=[
        pl.BlockSpec((BLK, D), lambda i: (i, 0)),
        pl.BlockSpec((2, D), lambda i: (0, 0)),
        pl.BlockSpec((1, D), lambda i: (0, 0)),
        pl.BlockSpec((1, D), lambda i: (0, 0)),
    ],
    out_specs=pl.BlockSpec((BLK, D), lambda i: (i, 0)),
    out_shape=jax.ShapeDtypeStruct((N, D), jnp.float32),
)


def _pool_body(hp_ref, stats_ref, g_ref, b_ref, bidx_ref,
               pw1_ref, pb1_ref, pw2_ref, pb2_ref,
               out_ref, accp_ref, accc_ref):
    i = pl.program_id(0)
    mean, inv = _bn_stats(stats_ref[...])
    h = jnp.maximum(
        (hp_ref[...] - mean) * inv * g_ref[...] + b_ref[...], 0.0)
    onehot_t = (lax.broadcasted_iota(jnp.int32, (G, BLK), 0)
                == bidx_ref[0]).astype(jnp.float32)
    pp = jnp.dot(onehot_t, h, preferred_element_type=jnp.float32)
    cnt = jnp.sum(onehot_t, axis=1, keepdims=True)
    pc = jnp.broadcast_to(cnt, (G, D))

    @pl.when(i == 0)
    def _():
        accp_ref[...] = pp
        accc_ref[...] = pc

    @pl.when(i > 0)
    def _():
        accp_ref[...] = accp_ref[...] + pp
        accc_ref[...] = accc_ref[...] + pc

    @pl.when(i == NBLK - 1)
    def _():
        pooled = accp_ref[...] / jnp.maximum(accc_ref[...], 1.0)
        a = jnp.maximum(
            jnp.dot(pooled, pw1_ref[...], preferred_element_type=jnp.float32)
            + pb1_ref[...], 0.0)
        out_ref[...] = (jnp.dot(a, pw2_ref[...],
                                preferred_element_type=jnp.float32)
                        + pb2_ref[...])


def _make_pool_call(T):
    return pl.pallas_call(
        _pool_body,
        grid=(NBLK,),
        in_specs=[
            pl.BlockSpec((BLK, D), lambda i: (i, 0)),
            pl.BlockSpec((2, D), lambda i: (0, 0)),
            pl.BlockSpec((1, D), lambda i: (0, 0)),
            pl.BlockSpec((1, D), lambda i: (0, 0)),
            pl.BlockSpec((1, 1, BLK), lambda i: (i, 0, 0)),
            pl.BlockSpec((D, D), lambda i: (0, 0)),
            pl.BlockSpec((1, D), lambda i: (0, 0)),
            pl.BlockSpec((D, T), lambda i: (0, 0)),
            pl.BlockSpec((1, T), lambda i: (0, 0)),
        ],
        out_specs=pl.BlockSpec((G, T), lambda i: (0, 0)),
        out_shape=jax.ShapeDtypeStruct((G, T), jnp.float32),
        scratch_shapes=[
            pltpu.VMEM((G, D), jnp.float32),
            pltpu.VMEM((G, D), jnp.float32),
        ],
    )


def kernel(x, edge_index, batch_idx,
           l0_W1, l0_b1, l0_W2, l0_b2, l0_gamma, l0_beta,
           l1_W1, l1_b1, l1_W2, l1_b2, l1_gamma, l1_beta,
           l2_W1, l2_b1, l2_W2, l2_b2, l2_gamma, l2_beta,
           p_W1, p_b1, p_W2, p_b2):
    T = p_W2.shape[1]
    src = edge_index[0]
    dst = edge_index[1]
    pad = EPAD - E
    src_p = jnp.concatenate(
        [src, jnp.zeros((pad,), jnp.int32)]).reshape(NW, CPT, CHUNK)
    dst_p = jnp.concatenate(
        [dst, jnp.full((pad,), N, jnp.int32)]).reshape(NW, CPT, CHUNK)
    zeros_blk = jnp.zeros((ROWS_PER_TILE, D), jnp.float32)
    bidx = batch_idx.reshape(NBLK, 1, BLK)

    layers = (
        (l0_W1, l0_b1, l0_W2, l0_b2, l0_gamma, l0_beta),
        (l1_W1, l1_b1, l1_W2, l1_b2, l1_gamma, l1_beta),
        (l2_W1, l2_b1, l2_W2, l2_b2, l2_gamma, l2_beta),
    )

    h = x
    for li, (W1, b1, W2, b2, gamma, beta) in enumerate(layers):
        p0, p1 = _sc_aggregate(h, src_p, dst_p, zeros_blk)
        hpre, stats = _mlp_call(h, p0, p1,
                                W1, b1.reshape(1, D), W2, b2.reshape(1, D))
        if li < 2:
            h = _bn_call(hpre, stats,
                         gamma.reshape(1, D), beta.reshape(1, D))
    out = _make_pool_call(T)(
        hpre, stats, l2_gamma.reshape(1, D), l2_beta.reshape(1, D), bidx,
        p_W1, p_b1.reshape(1, D), p_W2, p_b2.reshape(1, T))
    return out
